# baseline (speedup 1.0000x reference)
import functools

import jax
import jax.numpy as jnp
from jax.experimental import pallas as pl
from jax.sharding import Mesh, NamedSharding, PartitionSpec as P

try:
    from jax import shard_map as _shard_map
except ImportError:
    from jax.experimental.shard_map import shard_map as _shard_map

_K = 3


def _unfold_body(x_ref, d_ref, o_ref, *, cb, w, u_dim, l_dim, su):
    nstrips = u_dim // su
    n_groups = cb * _K * _K // 8
    for st in range(nstrips):
        u0 = st * su
        ds = d_ref[0, u0:u0 + su, :]
        is0 = (ds == 0)
        is1 = (ds == 1)
        fi = ((jax.lax.broadcasted_iota(jnp.int32, (su, l_dim), 0) + u0) * l_dim
              + jax.lax.broadcasted_iota(jnp.int32, (su, l_dim), 1))
        j = fi - (fi // w) * w
        col_ok = {s: (j + s >= 0) & (j + s < w) for s in (-1, 1, 3)}

        xs_cache = {}

        def get_xs(c):
            if c not in xs_cache:
                lo, hi = max(u0 - 2, 0), min(u0 + su + 6, u_dim)
                v = x_ref[0, c, lo:hi, :]
                if u0 - 2 < 0:
                    v = jnp.concatenate(
                        [jnp.zeros((2 - u0, l_dim), v.dtype), v], axis=0)
                if u0 + su + 6 > u_dim:
                    v = jnp.concatenate(
                        [v, jnp.zeros((u0 + su + 6 - u_dim, l_dim), v.dtype)],
                        axis=0)
                xs_cache[c] = v
            return xs_cache[c]

        v0_cache = {}

        def tap_value(c, r, s):
            xs = get_xs(c)
            q, m = divmod(r * w + s, l_dim)

            def sh(qq, mm):
                v = xs[2 + qq:2 + qq + su,
                       max(mm, 0):l_dim + min(mm, 0)]
                if mm > 0:
                    v = jnp.concatenate(
                        [v, jnp.zeros((su, mm), v.dtype)], axis=1)
                elif mm < 0:
                    v = jnp.concatenate(
                        [jnp.zeros((su, -mm), v.dtype), v], axis=1)
                return v

            v = sh(q, m) if m == 0 else sh(q, m) + sh(q + 1, m - l_dim)
            if s in col_ok:
                v = jnp.where(col_ok[s], v, 0.0)
            return v

        for g in range(n_groups):
            rows = []
            for sub in range(8):
                rl = 8 * g + sub
                c, t = rl // 9, rl % 9
                ki, kj = t // _K, t % _K
                if c not in v0_cache:
                    v0_cache[c] = tap_value(c, -1, -1)
                v1 = tap_value(c, ki - 1, kj - 1)
                v2 = tap_value(c, 2 * ki - 1, 2 * kj - 1)
                rows.append(
                    jnp.where(is0, v0_cache[c], jnp.where(is1, v1, v2)))
            t8 = jnp.stack(rows, axis=0).reshape(8, su * l_dim)
            o_ref[0, 8 * g:8 * g + 8, u0 * l_dim:(u0 + su) * l_dim] = t8


def _unfold_one_shard(xf, df, *, w, su):
    Bs, C, U, L = xf.shape
    F = U * L
    cb = 8
    return pl.pallas_call(
        functools.partial(_unfold_body, cb=cb, w=w, u_dim=U, l_dim=L, su=su),
        grid=(Bs, C // cb),
        in_specs=[
            pl.BlockSpec((1, cb, U, L), lambda b, c: (b, c, 0, 0)),
            pl.BlockSpec((1, U, L), lambda b, c: (b, 0, 0)),
        ],
        out_specs=pl.BlockSpec((1, cb * _K * _K, F), lambda b, c: (b, c, 0)),
        out_shape=jax.ShapeDtypeStruct((Bs, C * _K * _K, F), xf.dtype),
    )(xf, df)


@jax.jit
def kernel(input, dilation_map):
    B, C, H, W = input.shape
    F = H * W
    L = 128
    U = F // L

    devs = jax.devices()
    nd = 2 if len(devs) >= 2 and B % 2 == 0 else 1
    fn = functools.partial(_unfold_one_shard, w=W, su=56)
    if nd == 1:
        return fn(input.reshape(B, C, U, L), dilation_map.reshape(B, U, L))

    mesh = Mesh(devs[:nd], ("b",))
    sh = NamedSharding(mesh, P("b"))
    xf = jax.device_put(input, sh).reshape(B, C, U, L)
    df = jax.device_put(dilation_map, sh).reshape(B, U, L)
    out = _shard_map(fn, mesh=mesh, in_specs=(P("b"), P("b")),
                     out_specs=P("b"), check_vma=False)(xf, df)
    return out


# speedup vs baseline: 1.4989x; 1.4989x over previous
import functools

import jax
import jax.numpy as jnp
from jax.experimental import pallas as pl
_K = 3


def _unfold_body(x_ref, d_ref, o_ref, *, cb, w, u_dim, l_dim, su):
    nstrips = u_dim // su
    n_groups = cb * _K * _K // 8
    for st in range(nstrips):
        u0 = st * su
        ds = d_ref[0, u0:u0 + su, :]
        is0 = (ds == 0)
        is1 = (ds == 1)
        fi = ((jax.lax.broadcasted_iota(jnp.int32, (su, l_dim), 0) + u0) * l_dim
              + jax.lax.broadcasted_iota(jnp.int32, (su, l_dim), 1))
        j = fi - (fi // w) * w
        col_ok = {s: (j + s >= 0) & (j + s < w) for s in (-1, 1, 3)}

        xs_cache = {}

        def get_xs(c):
            if c not in xs_cache:
                lo, hi = max(u0 - 2, 0), min(u0 + su + 6, u_dim)
                v = x_ref[0, c, lo:hi, :]
                if u0 - 2 < 0:
                    v = jnp.concatenate(
                        [jnp.zeros((2 - u0, l_dim), v.dtype), v], axis=0)
                if u0 + su + 6 > u_dim:
                    v = jnp.concatenate(
                        [v, jnp.zeros((u0 + su + 6 - u_dim, l_dim), v.dtype)],
                        axis=0)
                xs_cache[c] = v
            return xs_cache[c]

        v0_cache = {}

        def tap_value(c, r, s):
            xs = get_xs(c)
            q, m = divmod(r * w + s, l_dim)

            def sh(qq, mm):
                v = xs[2 + qq:2 + qq + su,
                       max(mm, 0):l_dim + min(mm, 0)]
                if mm > 0:
                    v = jnp.concatenate(
                        [v, jnp.zeros((su, mm), v.dtype)], axis=1)
                elif mm < 0:
                    v = jnp.concatenate(
                        [jnp.zeros((su, -mm), v.dtype), v], axis=1)
                return v

            v = sh(q, m) if m == 0 else sh(q, m) + sh(q + 1, m - l_dim)
            if s in col_ok:
                v = jnp.where(col_ok[s], v, 0.0)
            return v

        for g in range(n_groups):
            rows = []
            for sub in range(8):
                rl = 8 * g + sub
                c, t = rl // 9, rl % 9
                ki, kj = t // _K, t % _K
                if c not in v0_cache:
                    v0_cache[c] = tap_value(c, -1, -1)
                v1 = tap_value(c, ki - 1, kj - 1)
                v2 = tap_value(c, 2 * ki - 1, 2 * kj - 1)
                rows.append(
                    jnp.where(is0, v0_cache[c], jnp.where(is1, v1, v2)))
            t8 = jnp.stack(rows, axis=0).reshape(8, su * l_dim)
            o_ref[0, 8 * g:8 * g + 8, u0 * l_dim:(u0 + su) * l_dim] = t8


def _unfold_one_shard(xf, df, *, w, su):
    Bs, C, U, L = xf.shape
    F = U * L
    cb = 8
    return pl.pallas_call(
        functools.partial(_unfold_body, cb=cb, w=w, u_dim=U, l_dim=L, su=su),
        grid=(Bs, C // cb),
        in_specs=[
            pl.BlockSpec((1, cb, U, L), lambda b, c: (b, c, 0, 0)),
            pl.BlockSpec((1, U, L), lambda b, c: (b, 0, 0)),
        ],
        out_specs=pl.BlockSpec((1, cb * _K * _K, F), lambda b, c: (b, c, 0)),
        out_shape=jax.ShapeDtypeStruct((Bs, C * _K * _K, F), xf.dtype),
    )(xf, df)


@jax.jit
def kernel(input, dilation_map):
    B, C, H, W = input.shape
    F = H * W
    L = 128
    U = F // L

    fn = functools.partial(_unfold_one_shard, w=W, su=28)
    return fn(input.reshape(B, C, U, L), dilation_map.reshape(B, U, L))


# speedup vs baseline: 1.5195x; 1.0137x over previous
import functools

import jax
import jax.numpy as jnp
from jax.experimental import pallas as pl
_K = 3


def _unfold_body(x_ref, d_ref, o_ref, *, cb, w, u_dim, l_dim, su):
    nstrips = u_dim // su
    n_groups = cb * _K * _K // 8
    for st in range(nstrips):
        u0 = st * su
        ds = d_ref[0, u0:u0 + su, :]
        is0 = (ds == 0)
        is1 = (ds == 1)
        fi = ((jax.lax.broadcasted_iota(jnp.int32, (su, l_dim), 0) + u0) * l_dim
              + jax.lax.broadcasted_iota(jnp.int32, (su, l_dim), 1))
        j = fi - (fi // w) * w
        col_ok = {s: (j + s >= 0) & (j + s < w) for s in (-1, 1, 3)}

        xs_cache = {}

        def get_xs(c):
            if c not in xs_cache:
                lo, hi = max(u0 - 2, 0), min(u0 + su + 6, u_dim)
                v = x_ref[0, c, lo:hi, :]
                if u0 - 2 < 0:
                    v = jnp.concatenate(
                        [jnp.zeros((2 - u0, l_dim), v.dtype), v], axis=0)
                if u0 + su + 6 > u_dim:
                    v = jnp.concatenate(
                        [v, jnp.zeros((u0 + su + 6 - u_dim, l_dim), v.dtype)],
                        axis=0)
                xs_cache[c] = v
            return xs_cache[c]

        v0_cache = {}

        def tap_value(c, r, s):
            xs = get_xs(c)
            q, m = divmod(r * w + s, l_dim)

            def sh(qq, mm):
                v = xs[2 + qq:2 + qq + su,
                       max(mm, 0):l_dim + min(mm, 0)]
                if mm > 0:
                    v = jnp.concatenate(
                        [v, jnp.zeros((su, mm), v.dtype)], axis=1)
                elif mm < 0:
                    v = jnp.concatenate(
                        [jnp.zeros((su, -mm), v.dtype), v], axis=1)
                return v

            v = sh(q, m) if m == 0 else sh(q, m) + sh(q + 1, m - l_dim)
            if s in col_ok:
                v = jnp.where(col_ok[s], v, 0.0)
            return v

        for g in range(n_groups):
            rows = []
            for sub in range(8):
                rl = 8 * g + sub
                c, t = rl // 9, rl % 9
                ki, kj = t // _K, t % _K
                if c not in v0_cache:
                    v0_cache[c] = tap_value(c, -1, -1)
                v1 = tap_value(c, ki - 1, kj - 1)
                v2 = tap_value(c, 2 * ki - 1, 2 * kj - 1)
                rows.append(
                    jnp.where(is0, v0_cache[c], jnp.where(is1, v1, v2)))
            t8 = jnp.stack(rows, axis=0).reshape(8, su * l_dim)
            o_ref[0, 8 * g:8 * g + 8, u0 * l_dim:(u0 + su) * l_dim] = t8


def _unfold_one_shard(xf, df, *, w, su):
    Bs, C, U, L = xf.shape
    F = U * L
    cb = 8
    return pl.pallas_call(
        functools.partial(_unfold_body, cb=cb, w=w, u_dim=U, l_dim=L, su=su),
        grid=(Bs, C // cb),
        in_specs=[
            pl.BlockSpec((1, cb, U, L), lambda b, c: (b, c, 0, 0)),
            pl.BlockSpec((1, U, L), lambda b, c: (b, 0, 0)),
        ],
        out_specs=pl.BlockSpec((1, cb * _K * _K, F), lambda b, c: (b, c, 0)),
        out_shape=jax.ShapeDtypeStruct((Bs, C * _K * _K, F), xf.dtype),
    )(xf, df)


@jax.jit
def kernel(input, dilation_map):
    B, C, H, W = input.shape
    F = H * W
    L = 128
    U = F // L

    fn = functools.partial(_unfold_one_shard, w=W, su=14)
    return fn(input.reshape(B, C, U, L), dilation_map.reshape(B, U, L))
